# baseline (device time: 39438 ns/iter reference)
import jax
import jax.numpy as jnp
from jax import lax
from jax.experimental import pallas as pl
from jax.experimental.pallas import tpu as pltpu

N_DEV = 4


def kernel(x, Win0, Wout0, Win1, Wout1, Win2, Wout2):
    B, D = x.shape
    H = Win0.shape[1]
    rows = B // N_DEV

    def body(x_ref, win0, wout0, win1, wout1, win2, wout2, out_ref,
             winbuf, woutbuf, win_bf, wout_bf,
             pstage, rstage, xstage, rsbuf, agbuf,
             win_sem, wout_sem,
             rs_send, rs_recv, ag_send, ag_recv):
        my = lax.axis_index("i")
        wins = [win0, win1, win2]
        wouts = [wout0, wout1, wout2]
        win_dma = [pltpu.make_async_copy(wins[r], winbuf, win_sem)
                   for r in range(3)]
        wout_dma = [pltpu.make_async_copy(wouts[r], woutbuf, wout_sem)
                    for r in range(3)]
        pending = []

        win_dma[0].start()
        wout_dma[0].start()

        barrier_sem = pltpu.get_barrier_semaphore()
        for k in range(1, N_DEV):
            pl.semaphore_signal(
                barrier_sem, inc=1,
                device_id=((my + k) % N_DEV,),
                device_id_type=pl.DeviceIdType.MESH,
            )
        pl.semaphore_wait(barrier_sem, N_DEV - 1)

        def load_weights(r):
            win_dma[r].wait()
            win_bf[...] = winbuf[...].astype(jnp.bfloat16)
            wout_dma[r].wait()
            wout_bf[...] = woutbuf[...].astype(jnp.bfloat16)

        def block(xrows_bf16):
            h = jnp.dot(xrows_bf16, win_bf[...],
                        preferred_element_type=jnp.float32)
            h = jnp.maximum(h, 0.0).astype(jnp.bfloat16)
            return jnp.dot(h, wout_bf[...], preferred_element_type=jnp.float32)

        xb = x_ref[...].astype(jnp.bfloat16)
        load_weights(0)
        p0 = block(xb)
        win_dma[1].start()
        wout_dma[1].start()
        pstage[...] = p0.astype(jnp.bfloat16)
        rs0 = []
        for k in range(1, N_DEV):
            dest = (my + k) % N_DEV
            r = pltpu.make_async_remote_copy(
                src_ref=pstage.at[pl.ds(dest * rows, rows)],
                dst_ref=rsbuf.at[0, k],
                send_sem=rs_send.at[0, k],
                recv_sem=rs_recv.at[0, k],
                device_id=(dest,),
                device_id_type=pl.DeviceIdType.MESH,
            )
            r.start()
            rs0.append(r)
        for r in rs0:
            r.wait_recv()
        pending += rs0
        x_mine = pstage[pl.ds(my * rows, rows)].astype(jnp.float32)
        for k in range(1, N_DEV):
            x_mine = x_mine + rsbuf[0, k].astype(jnp.float32)

        for r in (1, 2):
            ph = r - 1
            xstage[ph] = x_mine.astype(jnp.bfloat16)
            ag = []
            for k in range(1, N_DEV):
                dest = (my + k) % N_DEV
                a = pltpu.make_async_remote_copy(
                    src_ref=xstage.at[ph],
                    dst_ref=agbuf.at[ph, k],
                    send_sem=ag_send.at[ph, k],
                    recv_sem=ag_recv.at[ph, k],
                    device_id=(dest,),
                    device_id_type=pl.DeviceIdType.MESH,
                )
                a.start()
                ag.append(a)
            pending += ag

            load_weights(r)
            acc = block(x_mine.astype(jnp.bfloat16))

            rsr = []
            for k in range(1, N_DEV):
                ag[k - 1].wait_recv()
                pr = block(agbuf[ph, k])
                rstage[ph, k] = pr.astype(jnp.bfloat16)
                dest = (my - k) % N_DEV
                rd = pltpu.make_async_remote_copy(
                    src_ref=rstage.at[ph, k],
                    dst_ref=rsbuf.at[r, k],
                    send_sem=rs_send.at[r, k],
                    recv_sem=rs_recv.at[r, k],
                    device_id=(dest,),
                    device_id_type=pl.DeviceIdType.MESH,
                )
                rd.start()
                rsr.append(rd)
            if r < 2:
                win_dma[r + 1].start()
                wout_dma[r + 1].start()
            for rd in rsr:
                rd.wait_recv()
            pending += rsr
            x_mine = acc
            for k in range(1, N_DEV):
                x_mine = x_mine + rsbuf[r, k].astype(jnp.float32)

        out_ref[...] = x_mine
        for r in pending:
            r.wait_send()

    return pl.pallas_call(
        body,
        out_shape=jax.ShapeDtypeStruct((rows, D), jnp.float32),
        in_specs=[pl.BlockSpec(memory_space=pltpu.VMEM)]
        + [pl.BlockSpec(memory_space=pl.ANY)] * 6,
        out_specs=pl.BlockSpec(memory_space=pltpu.VMEM),
        scratch_shapes=[
            pltpu.VMEM((D, H), jnp.float32),
            pltpu.VMEM((H, D), jnp.float32),
            pltpu.VMEM((D, H), jnp.bfloat16),
            pltpu.VMEM((H, D), jnp.bfloat16),
            pltpu.VMEM((B, D), jnp.bfloat16),
            pltpu.VMEM((2, N_DEV, rows, D), jnp.bfloat16),
            pltpu.VMEM((2, rows, D), jnp.bfloat16),
            pltpu.VMEM((3, N_DEV, rows, D), jnp.bfloat16),
            pltpu.VMEM((2, N_DEV, rows, D), jnp.bfloat16),
            pltpu.SemaphoreType.DMA,
            pltpu.SemaphoreType.DMA,
            pltpu.SemaphoreType.DMA((3, N_DEV)),
            pltpu.SemaphoreType.DMA((3, N_DEV)),
            pltpu.SemaphoreType.DMA((2, N_DEV)),
            pltpu.SemaphoreType.DMA((2, N_DEV)),
        ],
        compiler_params=pltpu.CompilerParams(collective_id=0),
    )(x, Win0, Wout0, Win1, Wout1, Win2, Wout2)


# device time: 35979 ns/iter; 1.0961x vs baseline; 1.0961x over previous
import jax
import jax.numpy as jnp
from jax import lax
from jax.experimental import pallas as pl
from jax.experimental.pallas import tpu as pltpu

N_DEV = 4


def kernel(x, Win0, Wout0, Win1, Wout1, Win2, Wout2):
    B, D = x.shape
    H = Win0.shape[1]
    HH = H // 2
    rows = B // N_DEV

    def body(x_ref, win0, wout0, win1, wout1, win2, wout2, out_ref,
             xv, land, win_bf, wout_bf, ar_buf, rs_stage, rs_buf,
             x_sem, land_sems,
             ar_send_sems, ar_recv_sems, rs_send_sems, rs_recv_sems):
        my = lax.axis_index("i")
        wins = [win0, win1, win2]
        wouts = [wout0, wout1, wout2]

        def layer_chunks(r):
            srcs = [
                (wins[r].at[:, pl.ds(0, HH)], win_bf.at[:, pl.ds(0, HH)]),
                (wins[r].at[:, pl.ds(HH, HH)], win_bf.at[:, pl.ds(HH, HH)]),
                (wouts[r].at[pl.ds(0, HH), :], wout_bf.at[pl.ds(0, HH), :]),
                (wouts[r].at[pl.ds(HH, HH), :], wout_bf.at[pl.ds(HH, HH), :]),
            ]
            dmas = [
                pltpu.make_async_copy(src, land.at[i % 2], land_sems.at[i % 2])
                for i, (src, _) in enumerate(srcs)
            ]
            dmas[0].start()
            dmas[1].start()
            for i, (_, dst) in enumerate(srcs):
                dmas[i].wait()
                dst[...] = land[i % 2].astype(jnp.bfloat16)
                if i + 2 < 4:
                    dmas[i + 2].start()

        xcopy = pltpu.make_async_copy(x_ref, xv, x_sem)
        xcopy.start()

        barrier_sem = pltpu.get_barrier_semaphore()
        for k in range(1, N_DEV):
            pl.semaphore_signal(
                barrier_sem, inc=1,
                device_id=((my + k) % N_DEV,),
                device_id_type=pl.DeviceIdType.MESH,
            )
        pl.semaphore_wait(barrier_sem, N_DEV - 1)

        layer_chunks(0)
        xcopy.wait()
        xb = xv[...].astype(jnp.bfloat16)

        def mlp(xb):
            h = jnp.dot(xb, win_bf[...], preferred_element_type=jnp.float32)
            h = jnp.maximum(h, 0.0).astype(jnp.bfloat16)
            return jnp.dot(h, wout_bf[...], preferred_element_type=jnp.float32)

        for r in range(2):
            p = mlp(xb)
            ar_buf[r, 0] = p.astype(jnp.bfloat16)
            rdmas = []
            for k in range(1, N_DEV):
                rdma = pltpu.make_async_remote_copy(
                    src_ref=ar_buf.at[r, 0],
                    dst_ref=ar_buf.at[r, k],
                    send_sem=ar_send_sems.at[r, k],
                    recv_sem=ar_recv_sems.at[r, k],
                    device_id=((my + k) % N_DEV,),
                    device_id_type=pl.DeviceIdType.MESH,
                )
                rdma.start()
                rdmas.append(rdma)
            layer_chunks(r + 1)
            for rdma in rdmas:
                rdma.wait_recv()
            total = p
            for k in range(1, N_DEV):
                total = total + ar_buf[r, k].astype(jnp.float32)
            for rdma in rdmas:
                rdma.wait_send()
            xb = total.astype(jnp.bfloat16)

        p2 = mlp(xb)
        rs_stage[...] = p2.astype(jnp.bfloat16)
        rs_rdmas = []
        for k in range(1, N_DEV):
            dest = (my + k) % N_DEV
            rdma = pltpu.make_async_remote_copy(
                src_ref=rs_stage.at[pl.ds(dest * rows, rows)],
                dst_ref=rs_buf.at[k],
                send_sem=rs_send_sems.at[k],
                recv_sem=rs_recv_sems.at[k],
                device_id=(dest,),
                device_id_type=pl.DeviceIdType.MESH,
            )
            rdma.start()
            rs_rdmas.append(rdma)
        for rdma in rs_rdmas:
            rdma.wait_recv()
        total = rs_stage[pl.ds(my * rows, rows)].astype(jnp.float32)
        for k in range(1, N_DEV):
            total = total + rs_buf[k].astype(jnp.float32)
        for rdma in rs_rdmas:
            rdma.wait_send()
        out_ref[...] = total

    return pl.pallas_call(
        body,
        out_shape=jax.ShapeDtypeStruct((rows, D), jnp.float32),
        in_specs=[pl.BlockSpec(memory_space=pltpu.MemorySpace.HBM)] * 7,
        out_specs=pl.BlockSpec(memory_space=pltpu.VMEM),
        scratch_shapes=[
            pltpu.VMEM((B, D), jnp.float32),
            pltpu.VMEM((2, HH, D), jnp.float32),
            pltpu.VMEM((D, H), jnp.bfloat16),
            pltpu.VMEM((H, D), jnp.bfloat16),
            pltpu.VMEM((2, N_DEV, B, D), jnp.bfloat16),
            pltpu.VMEM((B, D), jnp.bfloat16),
            pltpu.VMEM((N_DEV, rows, D), jnp.bfloat16),
            pltpu.SemaphoreType.DMA,
            pltpu.SemaphoreType.DMA((2,)),
            pltpu.SemaphoreType.DMA((2, N_DEV)),
            pltpu.SemaphoreType.DMA((2, N_DEV)),
            pltpu.SemaphoreType.DMA((N_DEV,)),
            pltpu.SemaphoreType.DMA((N_DEV,)),
        ],
        compiler_params=pltpu.CompilerParams(collective_id=0),
    )(x, Win0, Wout0, Win1, Wout1, Win2, Wout2)


# device time: 31358 ns/iter; 1.2577x vs baseline; 1.1474x over previous
import jax
import jax.numpy as jnp
from jax import lax
from jax.experimental import pallas as pl
from jax.experimental.pallas import tpu as pltpu

N_DEV = 4


def kernel(x, Win0, Wout0, Win1, Wout1, Win2, Wout2):
    B, D = x.shape
    H = Win0.shape[1]
    rows = B // N_DEV

    def body(x_ref, win0, wout0, win1, wout1, win2, wout2, out_ref,
             winbuf, woutbuf, win_bf, ar_buf, rs_stage, rs_buf,
             win_sem, wout_sem,
             ar_send_sems, ar_recv_sems, rs_send_sems, rs_recv_sems):
        my = lax.axis_index("i")
        wins = [win0, win1, win2]
        wouts = [wout0, wout1, wout2]
        win_dma = [pltpu.make_async_copy(wins[r], winbuf, win_sem)
                   for r in range(3)]
        wout_dma = [pltpu.make_async_copy(wouts[r], woutbuf, wout_sem)
                    for r in range(3)]

        win_dma[0].start()
        wout_dma[0].start()

        barrier_sem = pltpu.get_barrier_semaphore()
        for k in range(1, N_DEV):
            pl.semaphore_signal(
                barrier_sem, inc=1,
                device_id=((my + k) % N_DEV,),
                device_id_type=pl.DeviceIdType.MESH,
            )
        pl.semaphore_wait(barrier_sem, N_DEV - 1)

        def stage_weights(r):
            win_dma[r].wait()
            win_bf[...] = winbuf[...].astype(jnp.bfloat16)
            if r + 1 < 3:
                win_dma[r + 1].start()
            wout_dma[r].wait()

        def mlp(xb, r):
            h = jnp.dot(xb, win_bf[...], preferred_element_type=jnp.float32)
            h = jnp.maximum(h, 0.0).astype(jnp.bfloat16)
            p = jnp.dot(h, woutbuf[...].astype(jnp.bfloat16),
                        preferred_element_type=jnp.float32)
            if r + 1 < 3:
                wout_dma[r + 1].start()
            return p

        stage_weights(0)
        xb = x_ref[...].astype(jnp.bfloat16)

        for r in range(2):
            p = mlp(xb, r)
            ar_buf[r, 0] = p.astype(jnp.bfloat16)
            rdmas = []
            for k in range(1, N_DEV):
                rdma = pltpu.make_async_remote_copy(
                    src_ref=ar_buf.at[r, 0],
                    dst_ref=ar_buf.at[r, k],
                    send_sem=ar_send_sems.at[r, k],
                    recv_sem=ar_recv_sems.at[r, k],
                    device_id=((my + k) % N_DEV,),
                    device_id_type=pl.DeviceIdType.MESH,
                )
                rdma.start()
                rdmas.append(rdma)
            stage_weights(r + 1)
            for rdma in rdmas:
                rdma.wait_recv()
            total = p
            for k in range(1, N_DEV):
                total = total + ar_buf[r, k].astype(jnp.float32)
            for rdma in rdmas:
                rdma.wait_send()
            xb = total.astype(jnp.bfloat16)

        p2 = mlp(xb, 2)
        rs_stage[...] = p2.astype(jnp.bfloat16)
        rs_rdmas = []
        for k in range(1, N_DEV):
            dest = (my + k) % N_DEV
            rdma = pltpu.make_async_remote_copy(
                src_ref=rs_stage.at[pl.ds(dest * rows, rows)],
                dst_ref=rs_buf.at[k],
                send_sem=rs_send_sems.at[k],
                recv_sem=rs_recv_sems.at[k],
                device_id=(dest,),
                device_id_type=pl.DeviceIdType.MESH,
            )
            rdma.start()
            rs_rdmas.append(rdma)
        for rdma in rs_rdmas:
            rdma.wait_recv()
        total = rs_stage[pl.ds(my * rows, rows)].astype(jnp.float32)
        for k in range(1, N_DEV):
            total = total + rs_buf[k].astype(jnp.float32)
        for rdma in rs_rdmas:
            rdma.wait_send()
        out_ref[...] = total

    return pl.pallas_call(
        body,
        out_shape=jax.ShapeDtypeStruct((rows, D), jnp.float32),
        in_specs=[pl.BlockSpec(memory_space=pltpu.VMEM)]
        + [pl.BlockSpec(memory_space=pl.ANY)] * 6,
        out_specs=pl.BlockSpec(memory_space=pltpu.VMEM),
        scratch_shapes=[
            pltpu.VMEM((D, H), jnp.float32),
            pltpu.VMEM((H, D), jnp.float32),
            pltpu.VMEM((D, H), jnp.bfloat16),
            pltpu.VMEM((2, N_DEV, B, D), jnp.bfloat16),
            pltpu.VMEM((B, D), jnp.bfloat16),
            pltpu.VMEM((N_DEV, rows, D), jnp.bfloat16),
            pltpu.SemaphoreType.DMA,
            pltpu.SemaphoreType.DMA,
            pltpu.SemaphoreType.DMA((2, N_DEV)),
            pltpu.SemaphoreType.DMA((2, N_DEV)),
            pltpu.SemaphoreType.DMA((N_DEV,)),
            pltpu.SemaphoreType.DMA((N_DEV,)),
        ],
        compiler_params=pltpu.CompilerParams(collective_id=0),
    )(x, Win0, Wout0, Win1, Wout1, Win2, Wout2)
